# Initial kernel scaffold; baseline (speedup 1.0000x reference)
import jax
import jax.numpy as jnp
from jax.experimental import pallas as pl


def kernel(x, edge_index, pair_idxs_left, pair_idxs_right, y, W_lin, b_lin, W_gat, a_src, a_dst, b_gat, W1, b1, W2, b2, W3, b3):
    raise NotImplementedError("write your pallas kernel here")


# speedup vs baseline: 5.2788x; 5.2788x over previous
import functools

import jax
import jax.numpy as jnp
from jax import lax
from jax.experimental import pallas as pl
from jax.experimental.pallas import tpu as pltpu
from jax.experimental.pallas import tpu_sc as plsc

_NC = 2
_NS = 16
_NW = _NC * _NS


def _mlp_body(x_ref, w1_ref, b1_ref, w2_ref, b2_ref, w3_ref, b3_ref, o_ref):
    h = jnp.dot(x_ref[...], w1_ref[...], preferred_element_type=jnp.float32)
    h = jnp.maximum(h + b1_ref[...], 0.0)
    h = jnp.dot(h, w2_ref[...], preferred_element_type=jnp.float32) + b2_ref[...]
    o_ref[...] = jnp.dot(h, w3_ref[...], preferred_element_type=jnp.float32) + b3_ref[...]


def _mlp(x, W1, b1, W2, b2, W3, b3):
    n, d = x.shape
    h = W1.shape[1]
    h2 = W2.shape[1]
    out = W3.shape[1]
    blk = 2000
    grid = pl.cdiv(n, blk)
    return pl.pallas_call(
        _mlp_body,
        grid=(grid,),
        in_specs=[
            pl.BlockSpec((blk, d), lambda i: (i, 0)),
            pl.BlockSpec((d, h), lambda i: (0, 0)),
            pl.BlockSpec((1, h), lambda i: (0, 0)),
            pl.BlockSpec((h, h2), lambda i: (0, 0)),
            pl.BlockSpec((1, h2), lambda i: (0, 0)),
            pl.BlockSpec((h2, out), lambda i: (0, 0)),
            pl.BlockSpec((1, out), lambda i: (0, 0)),
        ],
        out_specs=pl.BlockSpec((blk, out), lambda i: (i, 0)),
        out_shape=jax.ShapeDtypeStruct((n, out), jnp.float32),
    )(x, W1, b1[None, :], W2, b2[None, :], W3, b3[None, :])


def _round_up(v, m):
    return (v + m - 1) // m * m


def _gather_pairs(xdnn, x, idx_l, idx_r):
    p = idx_l.shape[0]
    d = x.shape[1]
    cpw = _round_up(pl.cdiv(p, _NW), 8)
    chunk = 392
    nchunks = pl.cdiv(cpw, chunk)
    mesh = plsc.VectorSubcoreMesh(
        core_axis_name="c", subcore_axis_name="s",
        num_cores=_NC, num_subcores=_NS)

    @functools.partial(
        pl.kernel,
        out_type=[
            jax.ShapeDtypeStruct((2 * p, d), jnp.float32),
            jax.ShapeDtypeStruct((2 * p, d), jnp.float32),
        ],
        mesh=mesh,
        scratch_types=[
            pltpu.VMEM((cpw,), jnp.int32),
            pltpu.VMEM((cpw,), jnp.int32),
            pltpu.VMEM((chunk, d), jnp.float32),
            pltpu.VMEM((chunk, d), jnp.float32),
            pltpu.SemaphoreType.DMA,
            pltpu.SemaphoreType.DMA,
            pltpu.SemaphoreType.DMA,
            pltpu.SemaphoreType.DMA,
        ],
    )
    def k(xdnn_hbm, x_hbm, idxl_hbm, idxr_hbm, emb_out, feat_out,
          idxl_v, idxr_v, buf0, buf1, gsem0, gsem1, ssem0, ssem1):
        wid = lax.axis_index("s") * _NC + lax.axis_index("c")
        base = jnp.minimum(wid * cpw, p - cpw)
        pltpu.sync_copy(idxl_hbm.at[pl.ds(base, cpw)], idxl_v)
        pltpu.sync_copy(idxr_hbm.at[pl.ds(base, cpw)], idxr_v)

        bufs = (buf0, buf1)
        gsems = (gsem0, gsem1)
        ssems = (ssem0, ssem1)

        def off(j):
            return jnp.minimum(j * chunk, cpw - chunk)

        for table, idx_v, out, row0 in (
            (xdnn_hbm, idxl_v, emb_out, 0),
            (xdnn_hbm, idxr_v, emb_out, p),
            (x_hbm, idxl_v, feat_out, 0),
            (x_hbm, idxr_v, feat_out, p),
        ):
            def gather(j):
                return pltpu.async_copy(
                    table.at[idx_v.at[pl.ds(off(j), chunk)]],
                    bufs[j % 2], gsems[j % 2])

            def scatter(j):
                return pltpu.async_copy(
                    bufs[j % 2],
                    out.at[pl.ds(row0 + base + off(j), chunk)],
                    ssems[j % 2])

            g = gather(0)
            gets = [g]
            puts = []
            for j in range(nchunks):
                if j + 1 < nchunks:
                    if j >= 1:
                        puts[j - 1].wait()
                    gets.append(gather(j + 1))
                gets[j].wait()
                puts.append(scatter(j))
            if nchunks >= 2:
                puts[nchunks - 2].wait()
            puts[nchunks - 1].wait()

    emb, feat = k(xdnn, x, idx_l, idx_r)
    return emb.reshape(2, p, d), feat.reshape(2, p, d)


def kernel(x, edge_index, pair_idxs_left, pair_idxs_right, y,
           W_lin, b_lin, W_gat, a_src, a_dst, b_gat,
           W1, b1, W2, b2, W3, b3):
    x_dnn = _mlp(x, W1, b1, W2, b2, W3, b3)
    pair_embeddings, pair_features = _gather_pairs(
        x_dnn, x, pair_idxs_left, pair_idxs_right)
    return (pair_embeddings, pair_features, y)


# speedup vs baseline: 5.3545x; 1.0143x over previous
import functools

import jax
import jax.numpy as jnp
from jax import lax
from jax.experimental import pallas as pl
from jax.experimental.pallas import tpu as pltpu
from jax.experimental.pallas import tpu_sc as plsc

_NC = 2
_NS = 16
_NW = _NC * _NS


def _mlp_body(x_ref, w1_ref, b1_ref, w2_ref, b2_ref, w3_ref, b3_ref, o_ref):
    h = jnp.dot(x_ref[...], w1_ref[...], preferred_element_type=jnp.float32)
    h = jnp.maximum(h + b1_ref[...], 0.0)
    h = jnp.dot(h, w2_ref[...], preferred_element_type=jnp.float32) + b2_ref[...]
    o_ref[...] = jnp.dot(h, w3_ref[...], preferred_element_type=jnp.float32) + b3_ref[...]


def _mlp(x, W1, b1, W2, b2, W3, b3):
    n, d = x.shape
    h = W1.shape[1]
    h2 = W2.shape[1]
    out = W3.shape[1]
    blk = 2000
    grid = pl.cdiv(n, blk)
    return pl.pallas_call(
        _mlp_body,
        grid=(grid,),
        in_specs=[
            pl.BlockSpec((blk, d), lambda i: (i, 0)),
            pl.BlockSpec((d, h), lambda i: (0, 0)),
            pl.BlockSpec((1, h), lambda i: (0, 0)),
            pl.BlockSpec((h, h2), lambda i: (0, 0)),
            pl.BlockSpec((1, h2), lambda i: (0, 0)),
            pl.BlockSpec((h2, out), lambda i: (0, 0)),
            pl.BlockSpec((1, out), lambda i: (0, 0)),
        ],
        out_specs=pl.BlockSpec((blk, out), lambda i: (i, 0)),
        out_shape=jax.ShapeDtypeStruct((n, out), jnp.float32),
    )(x, W1, b1[None, :], W2, b2[None, :], W3, b3[None, :])


def _round_up(v, m):
    return (v + m - 1) // m * m


def _gather_pairs(xdnn, x, idx_l, idx_r):
    p = idx_l.shape[0]
    d = x.shape[1]
    cpw = _round_up(pl.cdiv(p, _NW), 8)
    chunk = 392
    nchunks = pl.cdiv(cpw, chunk)
    mesh = plsc.VectorSubcoreMesh(
        core_axis_name="c", subcore_axis_name="s",
        num_cores=_NC, num_subcores=_NS)

    @functools.partial(
        pl.kernel,
        out_type=[
            jax.ShapeDtypeStruct((2 * p, d), jnp.float32),
            jax.ShapeDtypeStruct((2 * p, d), jnp.float32),
        ],
        mesh=mesh,
        scratch_types=[
            pltpu.VMEM((cpw,), jnp.int32),
            pltpu.VMEM((cpw,), jnp.int32),
            pltpu.VMEM((chunk, d), jnp.float32),
            pltpu.VMEM((chunk, d), jnp.float32),
            pltpu.SemaphoreType.DMA,
            pltpu.SemaphoreType.DMA,
            pltpu.SemaphoreType.DMA,
            pltpu.SemaphoreType.DMA,
        ],
    )
    def k(xdnn_hbm, x_hbm, idxl_hbm, idxr_hbm, emb_out, feat_out,
          idxl_v, idxr_v, buf0, buf1, gsem0, gsem1, ssem0, ssem1):
        wid = lax.axis_index("s") * _NC + lax.axis_index("c")
        base = jnp.minimum(wid * cpw, p - cpw)
        pltpu.sync_copy(idxl_hbm.at[pl.ds(base, cpw)], idxl_v)
        pltpu.sync_copy(idxr_hbm.at[pl.ds(base, cpw)], idxr_v)

        bufs = (buf0, buf1)
        gsems = (gsem0, gsem1)
        ssems = (ssem0, ssem1)

        def off(j):
            return jnp.minimum(j * chunk, cpw - chunk)

        units = []
        for table, idx_v, out, row0 in (
            (xdnn_hbm, idxl_v, emb_out, 0),
            (xdnn_hbm, idxr_v, emb_out, p),
            (x_hbm, idxl_v, feat_out, 0),
            (x_hbm, idxr_v, feat_out, p),
        ):
            for j in range(nchunks):
                units.append((table, idx_v, out, row0, j))
        nu = len(units)

        def gather(i):
            table, idx_v, _, _, j = units[i]
            return pltpu.async_copy(
                table.at[idx_v.at[pl.ds(off(j), chunk)]],
                bufs[i % 2], gsems[i % 2])

        def scatter(i):
            _, _, out, row0, j = units[i]
            return pltpu.async_copy(
                bufs[i % 2],
                out.at[pl.ds(row0 + base + off(j), chunk)],
                ssems[i % 2])

        gets = [gather(0)]
        puts = []
        for i in range(nu):
            if i + 1 < nu:
                if i >= 1:
                    puts[i - 1].wait()
                gets.append(gather(i + 1))
            gets[i].wait()
            puts.append(scatter(i))
        puts[nu - 2].wait()
        puts[nu - 1].wait()

    emb, feat = k(xdnn, x, idx_l, idx_r)
    return emb.reshape(2, p, d), feat.reshape(2, p, d)


def kernel(x, edge_index, pair_idxs_left, pair_idxs_right, y,
           W_lin, b_lin, W_gat, a_src, a_dst, b_gat,
           W1, b1, W2, b2, W3, b3):
    x_dnn = _mlp(x, W1, b1, W2, b2, W3, b3)
    pair_embeddings, pair_features = _gather_pairs(
        x_dnn, x, pair_idxs_left, pair_idxs_right)
    return (pair_embeddings, pair_features, y)


# speedup vs baseline: 7.4249x; 1.3867x over previous
import functools

import jax
import jax.numpy as jnp
from jax import lax
from jax.experimental import pallas as pl
from jax.experimental.pallas import tpu as pltpu
from jax.experimental.pallas import tpu_sc as plsc

_NC = 2
_NS = 16
_NW = _NC * _NS


def _mlp_body(x_ref, w1_ref, b1_ref, w2_ref, b2_ref, w3_ref, b3_ref, o_ref):
    h = jnp.dot(x_ref[...], w1_ref[...], preferred_element_type=jnp.float32)
    h = jnp.maximum(h + b1_ref[...], 0.0)
    h = jnp.dot(h, w2_ref[...], preferred_element_type=jnp.float32) + b2_ref[...]
    o_ref[...] = jnp.dot(h, w3_ref[...], preferred_element_type=jnp.float32) + b3_ref[...]


def _mlp(x, W1, b1, W2, b2, W3, b3):
    n, d = x.shape
    h = W1.shape[1]
    h2 = W2.shape[1]
    out = W3.shape[1]
    blk = 2000
    grid = pl.cdiv(n, blk)
    return pl.pallas_call(
        _mlp_body,
        grid=(grid,),
        in_specs=[
            pl.BlockSpec((blk, d), lambda i: (i, 0)),
            pl.BlockSpec((d, h), lambda i: (0, 0)),
            pl.BlockSpec((1, h), lambda i: (0, 0)),
            pl.BlockSpec((h, h2), lambda i: (0, 0)),
            pl.BlockSpec((1, h2), lambda i: (0, 0)),
            pl.BlockSpec((h2, out), lambda i: (0, 0)),
            pl.BlockSpec((1, out), lambda i: (0, 0)),
        ],
        out_specs=pl.BlockSpec((blk, out), lambda i: (i, 0)),
        out_shape=jax.ShapeDtypeStruct((n, out), jnp.float32),
    )(x, W1, b1[None, :], W2, b2[None, :], W3, b3[None, :])


def _round_up(v, m):
    return (v + m - 1) // m * m


def _gather_pairs(xdnn, x, idx_l, idx_r):
    p = idx_l.shape[0]
    d = x.shape[1]
    n = x.shape[0]
    cpw = _round_up(pl.cdiv(p, _NW), 8)
    chunk = 160
    nchunks = pl.cdiv(cpw, chunk)
    spw = _round_up(pl.cdiv(n, _NS), 8)
    mesh = plsc.VectorSubcoreMesh(
        core_axis_name="c", subcore_axis_name="s",
        num_cores=_NC, num_subcores=_NS)

    @functools.partial(
        pl.kernel,
        out_type=[
            jax.ShapeDtypeStruct((2 * p, d), jnp.float32),
            jax.ShapeDtypeStruct((2 * p, d), jnp.float32),
        ],
        mesh=mesh,
        scratch_types=[
            pltpu.VMEM((cpw,), jnp.int32),
            pltpu.VMEM((cpw,), jnp.int32),
            pltpu.VMEM_SHARED((n, d), jnp.float32),
            pltpu.VMEM((chunk, d), jnp.float32),
            pltpu.VMEM((chunk, d), jnp.float32),
            pltpu.SemaphoreType.DMA,
            pltpu.SemaphoreType.DMA,
            pltpu.SemaphoreType.DMA,
            pltpu.SemaphoreType.DMA,
        ],
    )
    def k(xdnn_hbm, x_hbm, idxl_hbm, idxr_hbm, emb_out, feat_out,
          idxl_v, idxr_v, staged, buf0, buf1, gsem0, gsem1, ssem0, ssem1):
        cid = lax.axis_index("c")
        sid = lax.axis_index("s")
        wid = sid * _NC + cid
        base = jnp.minimum(wid * cpw, p - cpw)
        pltpu.sync_copy(idxl_hbm.at[pl.ds(base, cpw)], idxl_v)
        pltpu.sync_copy(idxr_hbm.at[pl.ds(base, cpw)], idxr_v)

        bufs = (buf0, buf1)
        gsems = (gsem0, gsem1)
        ssems = (ssem0, ssem1)
        sbase = jnp.minimum(sid * spw, n - spw)

        def off(j):
            return jnp.minimum(j * chunk, cpw - chunk)

        def run_phase(table_hbm, out):
            pltpu.sync_copy(table_hbm.at[pl.ds(sbase, spw)],
                            staged.at[pl.ds(sbase, spw)])
            plsc.subcore_barrier()

            units = []
            for idx_v, row0 in ((idxl_v, 0), (idxr_v, p)):
                for j in range(nchunks):
                    units.append((idx_v, row0, j))
            nu = len(units)

            def gather(i):
                idx_v, _, j = units[i]
                return pltpu.async_copy(
                    staged.at[idx_v.at[pl.ds(off(j), chunk)]],
                    bufs[i % 2], gsems[i % 2])

            def scatter(i):
                _, row0, j = units[i]
                return pltpu.async_copy(
                    bufs[i % 2],
                    out.at[pl.ds(row0 + base + off(j), chunk)],
                    ssems[i % 2])

            gets = [gather(0)]
            puts = []
            for i in range(nu):
                if i + 1 < nu:
                    if i >= 1:
                        puts[i - 1].wait()
                    gets.append(gather(i + 1))
                gets[i].wait()
                puts.append(scatter(i))
            puts[nu - 2].wait()
            puts[nu - 1].wait()
            plsc.subcore_barrier()

        run_phase(xdnn_hbm, emb_out)
        run_phase(x_hbm, feat_out)

    emb, feat = k(xdnn, x, idx_l, idx_r)
    return emb.reshape(2, p, d), feat.reshape(2, p, d)


def kernel(x, edge_index, pair_idxs_left, pair_idxs_right, y,
           W_lin, b_lin, W_gat, a_src, a_dst, b_gat,
           W1, b1, W2, b2, W3, b3):
    x_dnn = _mlp(x, W1, b1, W2, b2, W3, b3)
    pair_embeddings, pair_features = _gather_pairs(
        x_dnn, x, pair_idxs_left, pair_idxs_right)
    return (pair_embeddings, pair_features, y)
